# baseline (speedup 1.0000x reference)
import functools

import jax
import jax.numpy as jnp
from jax import lax
from jax.experimental import pallas as pl
from jax.experimental.pallas import tpu as pltpu
from jax.experimental.pallas import tpu_sc as plsc

N = 10000
E = 320000
D = 128
H1 = 64
OUT = 16
MH = 128
MO = 1

NC = 2
NS = 16
NW = NC * NS
L = 16

Np = 10240
RPS = Np // NS
BC = 128
KB = 2
CHB = KB * BC
EWR = 10240
NG = EWR // CHB
NBUF = 3
TW = NW - 1

NB = 256
FB = 6400

_GDN = lax.GatherDimensionNumbers(offset_dims=(), collapsed_slice_dims=(0,),
                                  start_index_map=(0,))


def _lane_broadcast(v, lane):
  idx = jnp.full((L, 1), lane, jnp.int32)
  return lax.gather(v, idx, _GDN, (1,),
                    mode=lax.GatherScatterMode.PROMISE_IN_BOUNDS)


def _leaky_exp(e):
  return jnp.exp(jnp.where(e > 0, e, 0.2 * e))


def _edge_kernel(F):
  mesh = plsc.VectorSubcoreMesh(core_axis_name="c", subcore_axis_name="s",
                                num_cores=NC, num_subcores=NS)

  @functools.partial(
      pl.kernel,
      mesh=mesh,
      compiler_params=pltpu.CompilerParams(needs_layout_passes=False,
                                           use_tc_tiling_on_sc=False),
      out_type=[jax.ShapeDtypeStruct((NC, Np, F), jnp.float32),
                jax.ShapeDtypeStruct((NC, Np), jnp.float32)],
      scratch_types=[
          pltpu.VMEM((NBUF, KB, BC), jnp.int32),
          pltpu.VMEM((NBUF, KB, BC), jnp.int32),
          pltpu.VMEM((Np,), jnp.float32),
          pltpu.VMEM((Np,), jnp.float32),
          pltpu.VMEM((NBUF, CHB), jnp.float32),
          pltpu.VMEM((NBUF, CHB, F), jnp.float32),
          pltpu.VMEM_SHARED((Np, F), jnp.float32),
          pltpu.VMEM_SHARED((Np,), jnp.float32),
          [pltpu.SemaphoreType.DMA] * NBUF,
          [pltpu.SemaphoreType.DMA] * NBUF,
          [pltpu.SemaphoreType.DMA] * NBUF,
      ])
  def k(srcf_hbm, dstf_hbm, tsrc_hbm, tdst_hbm, ssrc_hbm, sdst_hbm, h_hbm,
        out_hbm, den_hbm,
        src_v, dst_v, ssrc_v, sdst_v, ex_v, rows_v, out_sh, den_sh,
        gsem, rsem, dsem):
    c = lax.axis_index("c")
    s = lax.axis_index("s")
    wid = s * NC + c
    zero16 = jnp.full((L,), 0.0, jnp.float32)

    def zero_rows(i, carry):
      for g in range(F // L):
        rows_v[0, i, pl.ds(g * L, L)] = zero16
      return carry
    lax.fori_loop(0, CHB, zero_rows, 0)

    def zero_ex(i, carry):
      ex_v[0, pl.ds(i * L, L)] = zero16
      return carry
    lax.fori_loop(0, CHB // L, zero_ex, 0)

    for j in range(RPS // BC):
      pltpu.sync_copy(rows_v.at[0, pl.ds(0, BC)],
                      out_sh.at[pl.ds(s * RPS + j * BC, BC)])
      pltpu.sync_copy(ex_v.at[0, pl.ds(0, BC)],
                      den_sh.at[pl.ds(s * RPS + j * BC, BC)])

    pltpu.sync_copy(ssrc_hbm, ssrc_v)
    pltpu.sync_copy(sdst_hbm, sdst_v)

    plsc.subcore_barrier()

    def stage_front(g, r):
      @pl.when(wid < TW)
      def _():
        for b in range(KB):
          off = wid * EWR + g * CHB + b * BC
          pltpu.sync_copy(srcf_hbm.at[pl.ds(off, BC)], src_v.at[r, b])
          pltpu.sync_copy(dstf_hbm.at[pl.ds(off, BC)], dst_v.at[r, b])

      @pl.when(wid == TW)
      def _():
        for b in range(KB):
          off = g * CHB + b * BC
          pltpu.sync_copy(tsrc_hbm.at[pl.ds(off, BC)], src_v.at[r, b])
          pltpu.sync_copy(tdst_hbm.at[pl.ds(off, BC)], dst_v.at[r, b])

      for b in range(KB):
        for t in range(BC // L):
          sl = pl.ds(t * L, L)
          e = (plsc.load_gather(ssrc_v, [src_v[r, b, sl]])
               + plsc.load_gather(sdst_v, [dst_v[r, b, sl]]))
          ex_v[r, pl.ds(b * BC + t * L, L)] = _leaky_exp(e)
      for b in range(KB):
        pltpu.async_copy(ex_v.at[r, pl.ds(b * BC, BC)],
                         den_sh.at[dst_v.at[r, b]], dsem[r], add=True)
      for b in range(KB):
        pltpu.async_copy(h_hbm.at[src_v.at[r, b]],
                         rows_v.at[r, pl.ds(b * BC, BC)], gsem[r])

    def back(r):
      for b in range(KB):
        pltpu.make_async_copy(h_hbm.at[src_v.at[r, b]],
                              rows_v.at[r, pl.ds(b * BC, BC)],
                              gsem[r]).wait()

      def scale(eb, carry):
        exv = ex_v[r, pl.ds(eb * L, L)]
        for lane in range(L):
          w = _lane_broadcast(exv, lane)
          row = eb * L + lane
          for g2 in range(F // L):
            sl2 = pl.ds(g2 * L, L)
            rows_v[r, row, sl2] = rows_v[r, row, sl2] * w
        return carry
      lax.fori_loop(0, CHB // L, scale, 0)

      for b in range(KB):
        pltpu.async_copy(rows_v.at[r, pl.ds(b * BC, BC)],
                         out_sh.at[dst_v.at[r, b]], rsem[r], add=True)

    def drain_row(r):
      for b in range(KB):
        pltpu.make_async_copy(rows_v.at[r, pl.ds(b * BC, BC)],
                              out_sh.at[dst_v.at[r, b]], rsem[r]).wait()

    def drain_den(r):
      for b in range(KB):
        pltpu.make_async_copy(ex_v.at[r, pl.ds(b * BC, BC)],
                              den_sh.at[dst_v.at[r, b]], dsem[r]).wait()

    stage_front(0, 0)
    stage_front(1, 1)

    def body(g, carry):
      for r in range(NBUF):
        @pl.when(lax.rem(g, NBUF) == r)
        def _(r=r):
          back(r)
          r2 = (r + 2) % NBUF

          @pl.when(g < NG - 2)
          def _():
            @pl.when(g >= 1)
            def _():
              drain_den(r2)
              drain_row(r2)
            stage_front(g + 2, r2)
      return carry

    lax.fori_loop(0, NG, body, 0)

    for r in range(NBUF):
      drain_den(r)
      drain_row(r)

    plsc.subcore_barrier()

    pltpu.sync_copy(out_sh.at[pl.ds(s * RPS, RPS)],
                    out_hbm.at[c, pl.ds(s * RPS, RPS)])
    pltpu.sync_copy(den_sh.at[pl.ds(s * RPS, RPS)],
                    den_hbm.at[c, pl.ds(s * RPS, RPS)])

  return k


def _tc1_body(x_ref, w1_ref, as_ref, ad_ref, h1_ref, ss_ref, sd_ref):
  h = jnp.dot(x_ref[...], w1_ref[...], preferred_element_type=jnp.float32)
  h1_ref[...] = h
  ss_ref[...] = jnp.sum(h * as_ref[...], axis=1)
  sd_ref[...] = jnp.sum(h * ad_ref[...], axis=1)


def _tc2_body(op_ref, dn_ref, h1_ref, ss_ref, sd_ref, b1_ref, w2_ref,
              as2_ref, ad2_ref, h2_ref, ss2_ref, sd2_ref):
  exn = _leaky_exp(ss_ref[...] + sd_ref[...])
  h1v = h1_ref[...]
  p = op_ref[0] + op_ref[1] + exn[:, None] * h1v
  d = dn_ref[0] + dn_ref[1] + exn
  h = p / (d[:, None] + 1e-16) + b1_ref[...]
  h = jnp.maximum(h, 0.0)
  h2 = jnp.dot(h, w2_ref[...], preferred_element_type=jnp.float32)
  h2_ref[...] = h2
  ss2_ref[...] = jnp.sum(h2 * as2_ref[...], axis=1)
  sd2_ref[...] = jnp.sum(h2 * ad2_ref[...], axis=1)


def _tc2b_body(op_ref, dn_ref, h2_ref, ss2_ref, sd2_ref, b2_ref, o2_ref):
  exn = _leaky_exp(ss2_ref[...] + sd2_ref[...])
  p = op_ref[0] + op_ref[1] + exn[:, None] * h2_ref[...]
  d = dn_ref[0] + dn_ref[1] + exn
  o2_ref[...] = p / (d[:, None] + 1e-16) + b2_ref[...]


def _tc3_body(fl_ref, wm_ref, bm1_ref, wm2_ref, bm2_ref, o_ref, acc_ref):
  i = pl.program_id(0)

  @pl.when(i == 0)
  def _():
    acc_ref[...] = jnp.zeros_like(acc_ref)

  acc_ref[...] += jnp.dot(fl_ref[...], wm_ref[...],
                          preferred_element_type=jnp.float32,
                          precision=lax.Precision.HIGHEST)

  @pl.when(i == pl.num_programs(0) - 1)
  def _():
    hm = jnp.maximum(acc_ref[...] + bm1_ref[...], 0.0)
    o_ref[...] = (jnp.sum(hm * wm2_ref[...], axis=1, keepdims=True)
                  + bm2_ref[...])


def kernel(x, edge_index, W1, a_src1, a_dst1, b1, W2, a_src2, a_dst2, b2,
           Wm1, bm1, Wm2, bm2):
  ei = edge_index.astype(jnp.int32)
  npad = NW * EWR - E
  pad_idx = N + (jnp.arange(npad, dtype=jnp.int32) % (Np - N))
  tsrc = jnp.concatenate([ei[0, TW * EWR:], pad_idx])
  tdst = jnp.concatenate([ei[1, TW * EWR:], pad_idx])

  x_pad = jnp.pad(x, ((0, Np - N), (0, 0)))

  h1, s1s, s1d = pl.pallas_call(
      _tc1_body,
      grid=(Np // NB,),
      in_specs=[pl.BlockSpec((NB, D), lambda i: (i, 0)),
                pl.BlockSpec((D, H1), lambda i: (0, 0)),
                pl.BlockSpec((1, H1), lambda i: (0, 0)),
                pl.BlockSpec((1, H1), lambda i: (0, 0))],
      out_specs=[pl.BlockSpec((NB, H1), lambda i: (i, 0)),
                 pl.BlockSpec((NB,), lambda i: (i,)),
                 pl.BlockSpec((NB,), lambda i: (i,))],
      out_shape=[jax.ShapeDtypeStruct((Np, H1), jnp.float32),
                 jax.ShapeDtypeStruct((Np,), jnp.float32),
                 jax.ShapeDtypeStruct((Np,), jnp.float32)],
  )(x_pad, W1, a_src1.reshape(1, H1), a_dst1.reshape(1, H1))

  out1_p, den1_p = _edge_kernel(H1)(ei[0], ei[1], tsrc, tdst, s1s, s1d, h1)

  h2, s2s, s2d = pl.pallas_call(
      _tc2_body,
      grid=(Np // NB,),
      in_specs=[pl.BlockSpec((NC, NB, H1), lambda i: (0, i, 0)),
                pl.BlockSpec((NC, NB), lambda i: (0, i)),
                pl.BlockSpec((NB, H1), lambda i: (i, 0)),
                pl.BlockSpec((NB,), lambda i: (i,)),
                pl.BlockSpec((NB,), lambda i: (i,)),
                pl.BlockSpec((1, H1), lambda i: (0, 0)),
                pl.BlockSpec((H1, OUT), lambda i: (0, 0)),
                pl.BlockSpec((1, OUT), lambda i: (0, 0)),
                pl.BlockSpec((1, OUT), lambda i: (0, 0))],
      out_specs=[pl.BlockSpec((NB, OUT), lambda i: (i, 0)),
                 pl.BlockSpec((NB,), lambda i: (i,)),
                 pl.BlockSpec((NB,), lambda i: (i,))],
      out_shape=[jax.ShapeDtypeStruct((Np, OUT), jnp.float32),
                 jax.ShapeDtypeStruct((Np,), jnp.float32),
                 jax.ShapeDtypeStruct((Np,), jnp.float32)],
  )(out1_p, den1_p, h1, s1s, s1d, b1.reshape(1, H1), W2,
    a_src2.reshape(1, OUT), a_dst2.reshape(1, OUT))

  out2_p, den2_p = _edge_kernel(OUT)(ei[0], ei[1], tsrc, tdst, s2s, s2d, h2)

  o2 = pl.pallas_call(
      _tc2b_body,
      grid=(Np // NB,),
      in_specs=[pl.BlockSpec((NC, NB, OUT), lambda i: (0, i, 0)),
                pl.BlockSpec((NC, NB), lambda i: (0, i)),
                pl.BlockSpec((NB, OUT), lambda i: (i, 0)),
                pl.BlockSpec((NB,), lambda i: (i,)),
                pl.BlockSpec((NB,), lambda i: (i,)),
                pl.BlockSpec((1, OUT), lambda i: (0, 0))],
      out_specs=pl.BlockSpec((NB, OUT), lambda i: (i, 0)),
      out_shape=jax.ShapeDtypeStruct((Np, OUT), jnp.float32),
  )(out2_p, den2_p, h2, s2s, s2d, b2.reshape(1, OUT))

  flat = o2[:N].reshape(1, N * OUT)
  pred = pl.pallas_call(
      _tc3_body,
      grid=(N * OUT // FB,),
      in_specs=[pl.BlockSpec((1, FB), lambda i: (0, i)),
                pl.BlockSpec((FB, MH), lambda i: (i, 0)),
                pl.BlockSpec((1, MH), lambda i: (0, 0)),
                pl.BlockSpec((1, MH), lambda i: (0, 0)),
                pl.BlockSpec((1, 1), lambda i: (0, 0))],
      out_specs=pl.BlockSpec((1, 1), lambda i: (0, 0)),
      out_shape=jax.ShapeDtypeStruct((1, 1), jnp.float32),
      scratch_shapes=[pltpu.VMEM((1, MH), jnp.float32)],
  )(flat, Wm1, bm1.reshape(1, MH), Wm2.reshape(1, MH), bm2.reshape(1, 1))

  return pred.reshape(MO)


# speedup vs baseline: 1.2156x; 1.2156x over previous
import functools

import jax
import jax.numpy as jnp
from jax import lax
from jax.experimental import pallas as pl
from jax.experimental.pallas import tpu as pltpu
from jax.experimental.pallas import tpu_sc as plsc

N = 10000
E = 320000
D = 128
H1 = 64
OUT = 16
MH = 128
MO = 1

NC = 2
NS = 16
NW = NC * NS
L = 16

Np = 10240
RPS = Np // NS
BC = 128
KB = 2
CHB = KB * BC
EWR = 10240
NG = EWR // CHB
NBUF = 3
TW = NW - 1

NB = 256
FB = 6400

_GDN = lax.GatherDimensionNumbers(offset_dims=(), collapsed_slice_dims=(0,),
                                  start_index_map=(0,))


def _lane_broadcast(v, lane):
  idx = jnp.full((L, 1), lane, jnp.int32)
  return lax.gather(v, idx, _GDN, (1,),
                    mode=lax.GatherScatterMode.PROMISE_IN_BOUNDS)


def _leaky_exp(e):
  return jnp.exp(jnp.where(e > 0, e, 0.2 * e))


def _edge_kernel(F):
  mesh = plsc.VectorSubcoreMesh(core_axis_name="c", subcore_axis_name="s",
                                num_cores=NC, num_subcores=NS)

  @functools.partial(
      pl.kernel,
      mesh=mesh,
      compiler_params=pltpu.CompilerParams(needs_layout_passes=False,
                                           use_tc_tiling_on_sc=False),
      out_type=[jax.ShapeDtypeStruct((NC, Np, F), jnp.float32),
                jax.ShapeDtypeStruct((NC, Np), jnp.float32)],
      scratch_types=[
          pltpu.VMEM((NBUF, KB, BC), jnp.int32),
          pltpu.VMEM((NBUF, KB, BC), jnp.int32),
          pltpu.VMEM((2 * Np,), jnp.float32),
          pltpu.VMEM((NBUF, CHB), jnp.float32),
          pltpu.VMEM((NBUF, CHB, F), jnp.float32),
          pltpu.VMEM_SHARED((Np, F), jnp.float32),
          pltpu.VMEM_SHARED((Np,), jnp.float32),
          [pltpu.SemaphoreType.DMA] * NBUF,
          [pltpu.SemaphoreType.DMA] * NBUF,
          [pltpu.SemaphoreType.DMA] * NBUF,
          pltpu.SemaphoreType.DMA,
      ])
  def k(srcf_hbm, dstf_hbm, tsrc_hbm, tdst_hbm, sv_hbm, h_hbm,
        out_hbm, den_hbm,
        src_v, dst_v, sv_v, ex_v, rows_v, out_sh, den_sh,
        gsem, rsem, dsem, isem):
    c = lax.axis_index("c")
    s = lax.axis_index("s")
    wid = s * NC + c
    zero16 = jnp.full((L,), 0.0, jnp.float32)

    def zero_rows(i, carry):
      for g in range(F // L):
        rows_v[0, i, pl.ds(g * L, L)] = zero16
      return carry
    lax.fori_loop(0, CHB, zero_rows, 0)

    def zero_ex(i, carry):
      ex_v[0, pl.ds(i * L, L)] = zero16
      return carry
    lax.fori_loop(0, CHB // L, zero_ex, 0)

    for j in range(RPS // BC):
      pltpu.sync_copy(rows_v.at[0, pl.ds(0, BC)],
                      out_sh.at[pl.ds(s * RPS + j * BC, BC)])
      pltpu.sync_copy(ex_v.at[0, pl.ds(0, BC)],
                      den_sh.at[pl.ds(s * RPS + j * BC, BC)])

    pltpu.sync_copy(sv_hbm, sv_v)

    plsc.subcore_barrier()

    def stage_front(g, r):
      @pl.when(wid < TW)
      def _():
        row0 = wid * (EWR // BC) + g * KB
        c1 = pltpu.async_copy(srcf_hbm.at[pl.ds(row0, KB)], src_v.at[r], isem)
        c2 = pltpu.async_copy(dstf_hbm.at[pl.ds(row0, KB)], dst_v.at[r], isem)
        c1.wait()
        c2.wait()

      @pl.when(wid == TW)
      def _():
        c1 = pltpu.async_copy(tsrc_hbm.at[pl.ds(g * KB, KB)], src_v.at[r],
                              isem)
        c2 = pltpu.async_copy(tdst_hbm.at[pl.ds(g * KB, KB)], dst_v.at[r],
                              isem)
        c1.wait()
        c2.wait()

      for b in range(KB):
        for t in range(BC // L):
          sl = pl.ds(t * L, L)
          sidx = src_v[r, b, sl]
          didx = dst_v[r, b, sl]
          e = (plsc.load_gather(sv_v, [sidx + sidx])
               + plsc.load_gather(sv_v, [didx + didx + 1]))
          ex_v[r, pl.ds(b * BC + t * L, L)] = _leaky_exp(e)
      for b in range(KB):
        pltpu.async_copy(ex_v.at[r, pl.ds(b * BC, BC)],
                         den_sh.at[dst_v.at[r, b]], dsem[r], add=True)
      for b in range(KB):
        pltpu.async_copy(h_hbm.at[src_v.at[r, b]],
                         rows_v.at[r, pl.ds(b * BC, BC)], gsem[r])

    def back(r):
      for b in range(KB):
        pltpu.make_async_copy(h_hbm.at[src_v.at[r, b]],
                              rows_v.at[r, pl.ds(b * BC, BC)],
                              gsem[r]).wait()

      def scale(eb, carry):
        exv = ex_v[r, pl.ds(eb * L, L)]
        for lane in range(L):
          w = _lane_broadcast(exv, lane)
          row = eb * L + lane
          for g2 in range(F // L):
            sl2 = pl.ds(g2 * L, L)
            rows_v[r, row, sl2] = rows_v[r, row, sl2] * w
        return carry
      lax.fori_loop(0, CHB // L, scale, 0)

      for b in range(KB):
        pltpu.async_copy(rows_v.at[r, pl.ds(b * BC, BC)],
                         out_sh.at[dst_v.at[r, b]], rsem[r], add=True)

    def drain_row(r):
      for b in range(KB):
        pltpu.make_async_copy(rows_v.at[r, pl.ds(b * BC, BC)],
                              out_sh.at[dst_v.at[r, b]], rsem[r]).wait()

    def drain_den(r):
      for b in range(KB):
        pltpu.make_async_copy(ex_v.at[r, pl.ds(b * BC, BC)],
                              den_sh.at[dst_v.at[r, b]], dsem[r]).wait()

    stage_front(0, 0)
    stage_front(1, 1)

    def body(g, carry):
      for r in range(NBUF):
        @pl.when(lax.rem(g, NBUF) == r)
        def _(r=r):
          back(r)
          r2 = (r + 2) % NBUF

          @pl.when(g < NG - 2)
          def _():
            @pl.when(g >= 1)
            def _():
              drain_den(r2)
              drain_row(r2)
            stage_front(g + 2, r2)
      return carry

    lax.fori_loop(0, NG, body, 0)

    for r in range(NBUF):
      drain_den(r)
      drain_row(r)

    plsc.subcore_barrier()

    pltpu.sync_copy(out_sh.at[pl.ds(s * RPS, RPS)],
                    out_hbm.at[c, pl.ds(s * RPS, RPS)])
    pltpu.sync_copy(den_sh.at[pl.ds(s * RPS, RPS)],
                    den_hbm.at[c, pl.ds(s * RPS, RPS)])

  return k


def _tc1_body(x_ref, w1_ref, a_ref, h1_ref, sp_ref):
  h = jnp.dot(x_ref[...], w1_ref[...], preferred_element_type=jnp.float32)
  h1_ref[...] = h
  sp_ref[...] = jnp.dot(h, a_ref[...], preferred_element_type=jnp.float32)


def _tc2_body(op_ref, dn_ref, h1_ref, sp_ref, b1_ref, w2_ref,
              a2_ref, h2_ref, sp2_ref):
  spv = sp_ref[...]
  exn = _leaky_exp(spv[:, 0] + spv[:, 1])
  p = op_ref[0] + op_ref[1] + exn[:, None] * h1_ref[...]
  d = dn_ref[0] + dn_ref[1] + exn
  h = p / (d[:, None] + 1e-16) + b1_ref[...]
  h = jnp.maximum(h, 0.0)
  h2 = jnp.dot(h, w2_ref[...], preferred_element_type=jnp.float32)
  h2_ref[...] = h2
  sp2_ref[...] = jnp.dot(h2, a2_ref[...], preferred_element_type=jnp.float32)


def _tc2b_body(op_ref, dn_ref, h2_ref, sp2_ref, b2_ref, o2_ref):
  spv = sp2_ref[...]
  exn = _leaky_exp(spv[:, 0] + spv[:, 1])
  p = op_ref[0] + op_ref[1] + exn[:, None] * h2_ref[...]
  d = dn_ref[0] + dn_ref[1] + exn
  o2_ref[...] = p / (d[:, None] + 1e-16) + b2_ref[...]


def _tc3_body(fl_ref, wm_ref, bm1_ref, wm2_ref, bm2_ref, o_ref, acc_ref):
  i = pl.program_id(0)

  @pl.when(i == 0)
  def _():
    acc_ref[...] = jnp.zeros_like(acc_ref)

  acc_ref[...] += jnp.dot(fl_ref[...], wm_ref[...],
                          preferred_element_type=jnp.float32,
                          precision=lax.Precision.HIGHEST)

  @pl.when(i == pl.num_programs(0) - 1)
  def _():
    hm = jnp.maximum(acc_ref[...] + bm1_ref[...], 0.0)
    o_ref[...] = (jnp.sum(hm * wm2_ref[...], axis=1, keepdims=True)
                  + bm2_ref[...])


def kernel(x, edge_index, W1, a_src1, a_dst1, b1, W2, a_src2, a_dst2, b2,
           Wm1, bm1, Wm2, bm2):
  ei = edge_index.astype(jnp.int32)
  npad = NW * EWR - E
  pad_idx = N + (jnp.arange(npad, dtype=jnp.int32) % (Np - N))
  tsrc = jnp.concatenate([ei[0, TW * EWR:], pad_idx]).reshape(EWR // BC, BC)
  tdst = jnp.concatenate([ei[1, TW * EWR:], pad_idx]).reshape(EWR // BC, BC)
  srcf = ei[0].reshape(E // BC, BC)
  dstf = ei[1].reshape(E // BC, BC)

  x_pad = jnp.pad(x, ((0, Np - N), (0, 0)))
  A1 = jnp.stack([a_src1, a_dst1], axis=1)
  A2 = jnp.stack([a_src2, a_dst2], axis=1)

  h1, sp1 = pl.pallas_call(
      _tc1_body,
      grid=(Np // NB,),
      in_specs=[pl.BlockSpec((NB, D), lambda i: (i, 0)),
                pl.BlockSpec((D, H1), lambda i: (0, 0)),
                pl.BlockSpec((H1, 2), lambda i: (0, 0))],
      out_specs=[pl.BlockSpec((NB, H1), lambda i: (i, 0)),
                 pl.BlockSpec((NB, 2), lambda i: (i, 0))],
      out_shape=[jax.ShapeDtypeStruct((Np, H1), jnp.float32),
                 jax.ShapeDtypeStruct((Np, 2), jnp.float32)],
  )(x_pad, W1, A1)

  out1_p, den1_p = _edge_kernel(H1)(srcf, dstf, tsrc, tdst,
                                    sp1.reshape(2 * Np), h1)

  h2, sp2 = pl.pallas_call(
      _tc2_body,
      grid=(Np // NB,),
      in_specs=[pl.BlockSpec((NC, NB, H1), lambda i: (0, i, 0)),
                pl.BlockSpec((NC, NB), lambda i: (0, i)),
                pl.BlockSpec((NB, H1), lambda i: (i, 0)),
                pl.BlockSpec((NB, 2), lambda i: (i, 0)),
                pl.BlockSpec((1, H1), lambda i: (0, 0)),
                pl.BlockSpec((H1, OUT), lambda i: (0, 0)),
                pl.BlockSpec((OUT, 2), lambda i: (0, 0))],
      out_specs=[pl.BlockSpec((NB, OUT), lambda i: (i, 0)),
                 pl.BlockSpec((NB, 2), lambda i: (i, 0))],
      out_shape=[jax.ShapeDtypeStruct((Np, OUT), jnp.float32),
                 jax.ShapeDtypeStruct((Np, 2), jnp.float32)],
  )(out1_p, den1_p, h1, sp1, b1.reshape(1, H1), W2, A2)

  out2_p, den2_p = _edge_kernel(OUT)(srcf, dstf, tsrc, tdst,
                                     sp2.reshape(2 * Np), h2)

  o2 = pl.pallas_call(
      _tc2b_body,
      grid=(Np // NB,),
      in_specs=[pl.BlockSpec((NC, NB, OUT), lambda i: (0, i, 0)),
                pl.BlockSpec((NC, NB), lambda i: (0, i)),
                pl.BlockSpec((NB, OUT), lambda i: (i, 0)),
                pl.BlockSpec((NB, 2), lambda i: (i, 0)),
                pl.BlockSpec((1, OUT), lambda i: (0, 0))],
      out_specs=pl.BlockSpec((NB, OUT), lambda i: (i, 0)),
      out_shape=jax.ShapeDtypeStruct((Np, OUT), jnp.float32),
  )(out2_p, den2_p, h2, sp2, b2.reshape(1, OUT))

  flat = o2[:N].reshape(1, N * OUT)
  pred = pl.pallas_call(
      _tc3_body,
      grid=(N * OUT // FB,),
      in_specs=[pl.BlockSpec((1, FB), lambda i: (0, i)),
                pl.BlockSpec((FB, MH), lambda i: (i, 0)),
                pl.BlockSpec((1, MH), lambda i: (0, 0)),
                pl.BlockSpec((1, MH), lambda i: (0, 0)),
                pl.BlockSpec((1, 1), lambda i: (0, 0))],
      out_specs=pl.BlockSpec((1, 1), lambda i: (0, 0)),
      out_shape=jax.ShapeDtypeStruct((1, 1), jnp.float32),
      scratch_shapes=[pltpu.VMEM((1, MH), jnp.float32)],
  )(flat, Wm1, bm1.reshape(1, MH), Wm2.reshape(1, MH), bm2.reshape(1, 1))

  return pred.reshape(MO)
